# baseline (speedup 1.0000x reference)
import functools
import jax
import jax.numpy as jnp
from jax import lax
from jax.experimental import pallas as pl
from jax.experimental.pallas import tpu as pltpu
from jax.experimental.pallas import tpu_sc as plsc

N_EMBED = 100000
N_LAYER = 16
CH = 64
D = N_LAYER * CH
B_TOT = 16384
S = 4
LPS = N_LAYER // S
DS = LPS * CH
NC, NS, LANES = 2, 16, 16
NW = NC * NS
BPW = B_TOT // NW
CB = 32
NCHUNK = BPW // CB
WB = 128
NCHW = WB // CB
NWIN = BPW // WB
NBUF = 2

_mesh = plsc.VectorSubcoreMesh(core_axis_name="c", subcore_axis_name="s")


@functools.partial(
    pl.kernel,
    out_type=jax.ShapeDtypeStruct((DS, B_TOT), jnp.float32),
    mesh=_mesh,
    scratch_types=[
        pltpu.VMEM((BPW,), jnp.float32),
        [pltpu.VMEM((2 * CB,), jnp.int32) for _ in range(NBUF)],
        [pltpu.VMEM((CB,), jnp.float32) for _ in range(NBUF)],
        [pltpu.VMEM((CB,), jnp.float32) for _ in range(NBUF)],
        [pltpu.VMEM((2 * CB, DS), jnp.float32) for _ in range(NBUF)],
        [pltpu.VMEM((DS, WB), jnp.float32) for _ in range(NBUF)],
        [pltpu.SemaphoreType.DMA for _ in range(NBUF)],
        [pltpu.SemaphoreType.DMA for _ in range(NBUF)],
    ],
    compiler_params=pltpu.CompilerParams(needs_layout_passes=False),
)
def _embed_slice(ori_hbm, table_hbm, out_hbm,
                 ori_v, idxc, wl_v, wr_v, bufc, slab, gsem, osem):
    wid = lax.axis_index("s") * NC + lax.axis_index("c")
    base = wid * BPW
    pltpu.sync_copy(ori_hbm.at[pl.ds(base, BPW)], ori_v)

    lane_iota = lax.iota(jnp.int32, LANES)

    def stage_indices(q, b):
        for g in range(CB // LANES):
            o = ori_v[pl.ds(q * CB + g * LANES, LANES)]
            s = (o + 1.0) * 0.5 * float(N_EMBED)
            i0 = s.astype(jnp.int32)
            f = s - i0.astype(jnp.float32)
            il = jnp.where(i0 >= N_EMBED, i0 - N_EMBED, i0)
            i1 = i0 + 1
            ir = jnp.where(i1 >= N_EMBED, i1 - N_EMBED, i1)
            sl = pl.ds(g * LANES, LANES)
            idxc[b][sl] = il
            idxc[b][pl.ds(CB + g * LANES, LANES)] = ir
            wl_v[b][sl] = 1.0 - f
            wr_v[b][sl] = f

    def start_gather(b):
        pltpu.async_copy(table_hbm.at[idxc[b]], bufc[b], gsem[b])

    def wait_gather(b):
        pltpu.make_async_copy(table_hbm.at[idxc[b]], bufc[b], gsem[b]).wait()

    def interp_chunk(b, sw, col0):
        wlv = [wl_v[b][pl.ds(bg * LANES, LANES)] for bg in range(CB // LANES)]
        wrv = [wr_v[b][pl.ds(bg * LANES, LANES)] for bg in range(CB // LANES)]

        @plsc.parallel_loop(0, DS, unroll=8)
        def _dbody(dd):
            dv = jnp.zeros((LANES,), jnp.int32) + dd
            for bg in range(CB // LANES):
                left = plsc.load_gather(
                    bufc[b], [lane_iota + bg * LANES, dv])
                right = plsc.load_gather(
                    bufc[b], [lane_iota + (CB + bg * LANES), dv])
                val = left * wlv[bg] + right * wrv[bg]
                slab[sw][dd, pl.ds(col0 + bg * LANES, LANES)] = val

    stage_indices(0, 0)
    start_gather(0)

    def outer(w0, carry):
        for sw in range(NBUF):
            w = w0 + sw

            @pl.when(w >= NBUF)
            def _():
                pltpu.make_async_copy(
                    slab[sw], out_hbm.at[:, pl.ds(base, WB)], osem[sw]).wait()

            for c in range(NCHW):
                q = w * NCHW + c
                b = c % NBUF

                @pl.when(q + 1 < NCHUNK)
                def _():
                    stage_indices(q + 1, 1 - b)
                    start_gather(1 - b)

                wait_gather(b)
                interp_chunk(b, sw, c * CB)

            pltpu.async_copy(
                slab[sw], out_hbm.at[:, pl.ds(base + w * WB, WB)], osem[sw])
        return carry

    lax.fori_loop(0, NWIN // NBUF, lambda i, c: outer(i * NBUF, c), 0)

    for sw in range(NBUF):
        pltpu.make_async_copy(
            slab[sw], out_hbm.at[:, pl.ds(base, WB)], osem[sw]).wait()


def kernel(ori, embeds):
    slabs = []
    for si in range(S):
        table_s = embeds[:, si * LPS:(si + 1) * LPS, :].reshape(N_EMBED, DS)
        slabs.append(_embed_slice(ori, table_s))
    out_t = jnp.concatenate(slabs, axis=0)
    return out_t.reshape(N_LAYER, CH, B_TOT).transpose(2, 0, 1)


# speedup vs baseline: 1.2128x; 1.2128x over previous
import functools
import jax
import jax.numpy as jnp
from jax import lax
from jax.experimental import pallas as pl
from jax.experimental.pallas import tpu as pltpu
from jax.experimental.pallas import tpu_sc as plsc

N_EMBED = 100000
N_LAYER = 16
CH = 64
D = N_LAYER * CH
B_TOT = 16384
S = 4
LPS = N_LAYER // S
DS = LPS * CH
NC, NS, LANES = 2, 16, 16
NW = NC * NS
BPW = B_TOT // NW
CB = 64
NCHUNK = BPW // CB
NBUF = 2

_mesh = plsc.VectorSubcoreMesh(core_axis_name="c", subcore_axis_name="s")


@functools.partial(
    pl.kernel,
    out_type=jax.ShapeDtypeStruct((B_TOT, DS), jnp.float32),
    mesh=_mesh,
    scratch_types=[
        pltpu.VMEM((BPW,), jnp.float32),
        [pltpu.VMEM((2 * CB,), jnp.int32) for _ in range(NBUF)],
        [pltpu.VMEM((CB,), jnp.float32) for _ in range(NBUF)],
        [pltpu.VMEM((CB,), jnp.float32) for _ in range(NBUF)],
        [pltpu.VMEM((2 * CB, DS), jnp.float32) for _ in range(NBUF)],
        [pltpu.SemaphoreType.DMA for _ in range(NBUF)],
        [pltpu.SemaphoreType.DMA for _ in range(NBUF)],
    ],
    compiler_params=pltpu.CompilerParams(needs_layout_passes=False),
)
def _embed_slice(ori_hbm, table_hbm, out_hbm,
                 ori_v, idxc, wl_v, wr_v, bufc, gsem, osem):
    wid = lax.axis_index("s") * NC + lax.axis_index("c")
    base = wid * BPW
    pltpu.sync_copy(ori_hbm.at[pl.ds(base, BPW)], ori_v)

    def stage_indices(q, b):
        for g in range(CB // LANES):
            o = ori_v[pl.ds(q * CB + g * LANES, LANES)]
            s = (o + 1.0) * 0.5 * float(N_EMBED)
            i0 = s.astype(jnp.int32)
            f = s - i0.astype(jnp.float32)
            il = jnp.where(i0 >= N_EMBED, i0 - N_EMBED, i0)
            i1 = i0 + 1
            ir = jnp.where(i1 >= N_EMBED, i1 - N_EMBED, i1)
            sl = pl.ds(g * LANES, LANES)
            idxc[b][sl] = il
            idxc[b][pl.ds(CB + g * LANES, LANES)] = ir
            wl_v[b][sl] = 1.0 - f
            wr_v[b][sl] = f

    def start_gather(b):
        pltpu.async_copy(table_hbm.at[idxc[b]], bufc[b], gsem[b])

    def wait_gather(b):
        pltpu.make_async_copy(table_hbm.at[idxc[b]], bufc[b], gsem[b]).wait()

    def interp_chunk(b):
        def row_body(j, carry):
            jv = jnp.zeros((LANES,), jnp.int32) + j
            wl = plsc.load_gather(wl_v[b], [jv])
            wr = plsc.load_gather(wr_v[b], [jv])
            for v in range(DS // LANES):
                sl = pl.ds(v * LANES, LANES)
                bufc[b][j, sl] = bufc[b][j, sl] * wl + bufc[b][CB + j, sl] * wr
            return carry

        lax.fori_loop(0, CB, row_body, 0, unroll=2)

    stage_indices(0, 0)
    start_gather(0)

    def wait_outcopy(b):
        pltpu.make_async_copy(
            bufc[b].at[pl.ds(0, CB)],
            out_hbm.at[pl.ds(base, CB)], osem[b]).wait()

    def outer(q0, carry):
        for b in range(NBUF):
            q = q0 + b

            @pl.when(q + 1 < NCHUNK)
            def _():
                @pl.when(q >= 1)
                def _():
                    wait_outcopy(1 - b)

                stage_indices(q + 1, 1 - b)
                start_gather(1 - b)

            wait_gather(b)
            interp_chunk(b)
            pltpu.async_copy(
                bufc[b].at[pl.ds(0, CB)],
                out_hbm.at[pl.ds(base + q * CB, CB)], osem[b])
        return carry

    lax.fori_loop(0, NCHUNK // NBUF, lambda i, c: outer(i * NBUF, c), 0)

    for b in range(NBUF):
        wait_outcopy(b)


def kernel(ori, embeds):
    outs = []
    for si in range(S):
        table_s = embeds[:, si * LPS:(si + 1) * LPS, :].reshape(N_EMBED, DS)
        outs.append(_embed_slice(ori, table_s))
    out = jnp.concatenate(outs, axis=1)
    return out.reshape(B_TOT, N_LAYER, CH)


# speedup vs baseline: 1.5690x; 1.2937x over previous
import functools
import jax
import jax.numpy as jnp
from jax import lax
from jax.experimental import pallas as pl
from jax.experimental.pallas import tpu as pltpu
from jax.experimental.pallas import tpu_sc as plsc

N_EMBED = 100000
N_LAYER = 16
CH = 64
D = N_LAYER * CH
B_TOT = 16384
S = 4
LPS = N_LAYER // S
DS = LPS * CH
NC, NS, LANES = 2, 16, 16
NW = NC * NS
BPW = B_TOT // NW
CB = 64
NCHUNK = BPW // CB
NBUF = 2
BN = 1024

_mesh = plsc.VectorSubcoreMesh(core_axis_name="c", subcore_axis_name="s")


@functools.partial(
    pl.kernel,
    out_type=jax.ShapeDtypeStruct((B_TOT, DS), jnp.float32),
    mesh=_mesh,
    scratch_types=[
        pltpu.VMEM((BPW,), jnp.float32),
        [pltpu.VMEM((2 * CB,), jnp.int32) for _ in range(NBUF)],
        [pltpu.VMEM((CB,), jnp.float32) for _ in range(NBUF)],
        [pltpu.VMEM((CB,), jnp.float32) for _ in range(NBUF)],
        [pltpu.VMEM((2 * CB, DS), jnp.float32) for _ in range(NBUF)],
        [pltpu.SemaphoreType.DMA for _ in range(NBUF)],
        [pltpu.SemaphoreType.DMA for _ in range(NBUF)],
    ],
    compiler_params=pltpu.CompilerParams(needs_layout_passes=False),
)
def _embed_slice(ori_hbm, table_hbm, out_hbm,
                 ori_v, idxc, wl_v, wr_v, bufc, gsem, osem):
    wid = lax.axis_index("s") * NC + lax.axis_index("c")
    base = wid * BPW
    pltpu.sync_copy(ori_hbm.at[pl.ds(base, BPW)], ori_v)

    def stage_indices(q, b):
        for g in range(CB // LANES):
            o = ori_v[pl.ds(q * CB + g * LANES, LANES)]
            s = (o + 1.0) * 0.5 * float(N_EMBED)
            i0 = s.astype(jnp.int32)
            f = s - i0.astype(jnp.float32)
            il = jnp.where(i0 >= N_EMBED, i0 - N_EMBED, i0)
            i1 = i0 + 1
            ir = jnp.where(i1 >= N_EMBED, i1 - N_EMBED, i1)
            sl = pl.ds(g * LANES, LANES)
            idxc[b][sl] = il
            idxc[b][pl.ds(CB + g * LANES, LANES)] = ir
            wl_v[b][sl] = 1.0 - f
            wr_v[b][sl] = f

    def start_gather(b):
        pltpu.async_copy(table_hbm.at[idxc[b]], bufc[b], gsem[b])

    def wait_gather(b):
        pltpu.make_async_copy(table_hbm.at[idxc[b]], bufc[b], gsem[b]).wait()

    def interp_chunk(b):
        def row_body(j, carry):
            jv = jnp.zeros((LANES,), jnp.int32) + j
            wl = plsc.load_gather(wl_v[b], [jv])
            wr = plsc.load_gather(wr_v[b], [jv])
            for v in range(DS // LANES):
                sl = pl.ds(v * LANES, LANES)
                bufc[b][j, sl] = bufc[b][j, sl] * wl + bufc[b][CB + j, sl] * wr
            return carry

        lax.fori_loop(0, CB, row_body, 0, unroll=2)

    stage_indices(0, 0)
    start_gather(0)

    def wait_outcopy(b):
        pltpu.make_async_copy(
            bufc[b].at[pl.ds(0, CB)],
            out_hbm.at[pl.ds(base, CB)], osem[b]).wait()

    def outer(q0, carry):
        for b in range(NBUF):
            q = q0 + b

            @pl.when(q + 1 < NCHUNK)
            def _():
                @pl.when(q >= 1)
                def _():
                    wait_outcopy(1 - b)

                stage_indices(q + 1, 1 - b)
                start_gather(1 - b)

            wait_gather(b)
            interp_chunk(b)
            pltpu.async_copy(
                bufc[b].at[pl.ds(0, CB)],
                out_hbm.at[pl.ds(base + q * CB, CB)], osem[b])
        return carry

    lax.fori_loop(0, NCHUNK // NBUF, lambda i, c: outer(i * NBUF, c), 0)

    for b in range(NBUF):
        wait_outcopy(b)


def _transpose_body(x_ref, o_ref):
    x = x_ref[...]
    o_ref[...] = jnp.transpose(x.reshape(DS, BN), (1, 0))


def _make_transpose(si):
    grid = (N_EMBED + BN - 1) // BN
    return pl.pallas_call(
        _transpose_body,
        grid=(grid,),
        in_specs=[pl.BlockSpec((LPS, CH, BN), lambda i, si=si: (si, 0, i))],
        out_specs=pl.BlockSpec((BN, DS), lambda i: (i, 0)),
        out_shape=jax.ShapeDtypeStruct((N_EMBED, DS), jnp.float32),
    )


def kernel(ori, embeds):
    et = embeds.transpose(1, 2, 0)
    outs = []
    for si in range(S):
        table_s = _make_transpose(si)(et)
        outs.append(_embed_slice(ori, table_s))
    out = jnp.concatenate(outs, axis=1)
    return out.reshape(B_TOT, N_LAYER, CH)


# speedup vs baseline: 1.8979x; 1.2096x over previous
import functools
import jax
import jax.numpy as jnp
from jax import lax
from jax.experimental import pallas as pl
from jax.experimental.pallas import tpu as pltpu
from jax.experimental.pallas import tpu_sc as plsc

N_EMBED = 100000
N_LAYER = 16
CH = 64
D = N_LAYER * CH
B_TOT = 16384
S = 4
LPS = N_LAYER // S
DS = LPS * CH
NC, NS, LANES = 2, 16, 16
NW = NC * NS
BPW = B_TOT // NW
CB = 64
NCHUNK = BPW // CB
NBUF = 2
BN = 2048

_mesh = plsc.VectorSubcoreMesh(core_axis_name="c", subcore_axis_name="s")


@functools.partial(
    pl.kernel,
    out_type=jax.ShapeDtypeStruct((B_TOT, DS), jnp.float32),
    mesh=_mesh,
    scratch_types=[
        pltpu.VMEM((BPW,), jnp.float32),
        [pltpu.VMEM((2 * CB,), jnp.int32) for _ in range(NBUF)],
        [pltpu.VMEM((CB,), jnp.float32) for _ in range(NBUF)],
        [pltpu.VMEM((CB,), jnp.float32) for _ in range(NBUF)],
        [pltpu.VMEM((2 * CB, DS), jnp.float32) for _ in range(NBUF)],
        [pltpu.SemaphoreType.DMA for _ in range(NBUF)],
        [pltpu.SemaphoreType.DMA for _ in range(NBUF)],
    ],
    compiler_params=pltpu.CompilerParams(needs_layout_passes=False),
)
def _embed_slice(ori_hbm, table_hbm, out_hbm,
                 ori_v, idxc, wl_v, wr_v, bufc, gsem, osem):
    wid = lax.axis_index("s") * NC + lax.axis_index("c")
    base = wid * BPW
    pltpu.sync_copy(ori_hbm.at[pl.ds(base, BPW)], ori_v)

    def stage_indices(q, b):
        for g in range(CB // LANES):
            o = ori_v[pl.ds(q * CB + g * LANES, LANES)]
            s = (o + 1.0) * 0.5 * float(N_EMBED)
            i0 = s.astype(jnp.int32)
            f = s - i0.astype(jnp.float32)
            il = jnp.where(i0 >= N_EMBED, i0 - N_EMBED, i0)
            i1 = i0 + 1
            ir = jnp.where(i1 >= N_EMBED, i1 - N_EMBED, i1)
            sl = pl.ds(g * LANES, LANES)
            idxc[b][sl] = il
            idxc[b][pl.ds(CB + g * LANES, LANES)] = ir
            wl_v[b][sl] = 1.0 - f
            wr_v[b][sl] = f

    def start_gather(b):
        pltpu.async_copy(table_hbm.at[idxc[b]], bufc[b], gsem[b])

    def wait_gather(b):
        pltpu.make_async_copy(table_hbm.at[idxc[b]], bufc[b], gsem[b]).wait()

    def interp_chunk(b):
        def row_body(j, carry):
            jv = jnp.zeros((LANES,), jnp.int32) + j
            wl = plsc.load_gather(wl_v[b], [jv])
            wr = plsc.load_gather(wr_v[b], [jv])
            for v in range(DS // LANES):
                sl = pl.ds(v * LANES, LANES)
                bufc[b][j, sl] = bufc[b][j, sl] * wl + bufc[b][CB + j, sl] * wr
            return carry

        lax.fori_loop(0, CB, row_body, 0, unroll=2)

    stage_indices(0, 0)
    start_gather(0)

    def wait_outcopy(b):
        pltpu.make_async_copy(
            bufc[b].at[pl.ds(0, CB)],
            out_hbm.at[pl.ds(base, CB)], osem[b]).wait()

    def outer(q0, carry):
        for b in range(NBUF):
            q = q0 + b

            @pl.when(q + 1 < NCHUNK)
            def _():
                @pl.when(q >= 1)
                def _():
                    wait_outcopy(1 - b)

                stage_indices(q + 1, 1 - b)
                start_gather(1 - b)

            wait_gather(b)
            interp_chunk(b)
            pltpu.async_copy(
                bufc[b].at[pl.ds(0, CB)],
                out_hbm.at[pl.ds(base + q * CB, CB)], osem[b])
        return carry

    lax.fori_loop(0, NCHUNK // NBUF, lambda i, c: outer(i * NBUF, c), 0)

    for b in range(NBUF):
        wait_outcopy(b)


def _transpose_body(x_ref, o_ref):
    x = x_ref[...]
    o_ref[...] = jnp.transpose(x.reshape(DS, BN), (1, 0))


def _make_transpose(si):
    grid = (N_EMBED + BN - 1) // BN
    return pl.pallas_call(
        _transpose_body,
        grid=(grid,),
        in_specs=[pl.BlockSpec((LPS, CH, BN), lambda i, si=si: (si, 0, i))],
        out_specs=pl.BlockSpec((BN, DS), lambda i: (i, 0)),
        out_shape=jax.ShapeDtypeStruct((N_EMBED, DS), jnp.float32),
    )


def kernel(ori, embeds):
    et = embeds.transpose(1, 2, 0)
    outs = []
    for si in range(S):
        table_s = _make_transpose(si)(et)
        outs.append(_embed_slice(ori, table_s))
    out = jnp.concatenate(outs, axis=1)
    return out.reshape(B_TOT, N_LAYER, CH)


# speedup vs baseline: 2.0516x; 1.0810x over previous
import functools
import jax
import jax.numpy as jnp
from jax import lax
from jax.experimental import pallas as pl
from jax.experimental.pallas import tpu as pltpu
from jax.experimental.pallas import tpu_sc as plsc

N_EMBED = 100000
N_LAYER = 16
CH = 64
D = N_LAYER * CH
B_TOT = 16384
S = 4
LPS = N_LAYER // S
DS = LPS * CH
NC, NS, LANES = 2, 16, 16
NW = NC * NS
BPW = B_TOT // NW
CB = 64
NCHUNK = BPW // CB
NBUF = 2
BN = 4096

_mesh = plsc.VectorSubcoreMesh(core_axis_name="c", subcore_axis_name="s")


@functools.partial(
    pl.kernel,
    out_type=jax.ShapeDtypeStruct((B_TOT, DS), jnp.float32),
    mesh=_mesh,
    scratch_types=[
        pltpu.VMEM((BPW,), jnp.float32),
        [pltpu.VMEM((2 * CB,), jnp.int32) for _ in range(NBUF)],
        [pltpu.VMEM((CB,), jnp.float32) for _ in range(NBUF)],
        [pltpu.VMEM((CB,), jnp.float32) for _ in range(NBUF)],
        [pltpu.VMEM((2 * CB, DS), jnp.float32) for _ in range(NBUF)],
        [pltpu.SemaphoreType.DMA for _ in range(NBUF)],
        [pltpu.SemaphoreType.DMA for _ in range(NBUF)],
    ],
    compiler_params=pltpu.CompilerParams(needs_layout_passes=False),
)
def _embed_slice(ori_hbm, table_hbm, out_hbm,
                 ori_v, idxc, wl_v, wr_v, bufc, gsem, osem):
    wid = lax.axis_index("s") * NC + lax.axis_index("c")
    base = wid * BPW
    pltpu.sync_copy(ori_hbm.at[pl.ds(base, BPW)], ori_v)

    def stage_indices(q, b):
        for g in range(CB // LANES):
            o = ori_v[pl.ds(q * CB + g * LANES, LANES)]
            s = (o + 1.0) * 0.5 * float(N_EMBED)
            i0 = s.astype(jnp.int32)
            f = s - i0.astype(jnp.float32)
            il = jnp.where(i0 >= N_EMBED, i0 - N_EMBED, i0)
            i1 = i0 + 1
            ir = jnp.where(i1 >= N_EMBED, i1 - N_EMBED, i1)
            sl = pl.ds(g * LANES, LANES)
            idxc[b][sl] = il
            idxc[b][pl.ds(CB + g * LANES, LANES)] = ir
            wl_v[b][sl] = 1.0 - f
            wr_v[b][sl] = f

    def start_gather(b):
        pltpu.async_copy(table_hbm.at[idxc[b]], bufc[b], gsem[b])

    def wait_gather(b):
        pltpu.make_async_copy(table_hbm.at[idxc[b]], bufc[b], gsem[b]).wait()

    def interp_chunk(b):
        def row_body(j, carry):
            jv = jnp.zeros((LANES,), jnp.int32) + j
            wl = plsc.load_gather(wl_v[b], [jv])
            wr = plsc.load_gather(wr_v[b], [jv])
            for v in range(DS // LANES):
                sl = pl.ds(v * LANES, LANES)
                bufc[b][j, sl] = bufc[b][j, sl] * wl + bufc[b][CB + j, sl] * wr
            return carry

        lax.fori_loop(0, CB, row_body, 0, unroll=2)

    stage_indices(0, 0)
    start_gather(0)

    def wait_outcopy(b):
        pltpu.make_async_copy(
            bufc[b].at[pl.ds(0, CB)],
            out_hbm.at[pl.ds(base, CB)], osem[b]).wait()

    def outer(q0, carry):
        for b in range(NBUF):
            q = q0 + b

            @pl.when(q + 1 < NCHUNK)
            def _():
                @pl.when(q >= 1)
                def _():
                    wait_outcopy(1 - b)

                stage_indices(q + 1, 1 - b)
                start_gather(1 - b)

            wait_gather(b)
            interp_chunk(b)
            pltpu.async_copy(
                bufc[b].at[pl.ds(0, CB)],
                out_hbm.at[pl.ds(base + q * CB, CB)], osem[b])
        return carry

    lax.fori_loop(0, NCHUNK // NBUF, lambda i, c: outer(i * NBUF, c), 0)

    for b in range(NBUF):
        wait_outcopy(b)


def _transpose_body(x_ref, o_ref):
    x = x_ref[...]
    o_ref[...] = jnp.transpose(x.reshape(DS, BN), (1, 0))


def _make_transpose(si):
    grid = (N_EMBED + BN - 1) // BN
    return pl.pallas_call(
        _transpose_body,
        grid=(grid,),
        in_specs=[pl.BlockSpec((LPS, CH, BN), lambda i, si=si: (si, 0, i))],
        out_specs=pl.BlockSpec((BN, DS), lambda i: (i, 0)),
        out_shape=jax.ShapeDtypeStruct((N_EMBED, DS), jnp.float32),
    )


def kernel(ori, embeds):
    et = embeds.transpose(1, 2, 0)
    outs = []
    for si in range(S):
        table_s = _make_transpose(si)(et)
        outs.append(_embed_slice(ori, table_s))
    out = jnp.concatenate(outs, axis=1)
    return out.reshape(B_TOT, N_LAYER, CH)


# speedup vs baseline: 2.0878x; 1.0176x over previous
import functools
import jax
import jax.numpy as jnp
from jax import lax
from jax.experimental import pallas as pl
from jax.experimental.pallas import tpu as pltpu
from jax.experimental.pallas import tpu_sc as plsc

N_EMBED = 100000
N_LAYER = 16
CH = 64
D = N_LAYER * CH
B_TOT = 16384
S = 4
LPS = N_LAYER // S
DS = LPS * CH
NC, NS, LANES = 2, 16, 16
NW = NC * NS
BPW = B_TOT // NW
CB = 64
NCHUNK = BPW // CB
NBUF = 2
BN = 8192

_mesh = plsc.VectorSubcoreMesh(core_axis_name="c", subcore_axis_name="s")


@functools.partial(
    pl.kernel,
    out_type=jax.ShapeDtypeStruct((B_TOT, DS), jnp.float32),
    mesh=_mesh,
    scratch_types=[
        pltpu.VMEM((BPW,), jnp.float32),
        [pltpu.VMEM((2 * CB,), jnp.int32) for _ in range(NBUF)],
        [pltpu.VMEM((CB,), jnp.float32) for _ in range(NBUF)],
        [pltpu.VMEM((CB,), jnp.float32) for _ in range(NBUF)],
        [pltpu.VMEM((2 * CB, DS), jnp.float32) for _ in range(NBUF)],
        [pltpu.SemaphoreType.DMA for _ in range(NBUF)],
        [pltpu.SemaphoreType.DMA for _ in range(NBUF)],
    ],
    compiler_params=pltpu.CompilerParams(needs_layout_passes=False),
)
def _embed_slice(ori_hbm, table_hbm, out_hbm,
                 ori_v, idxc, wl_v, wr_v, bufc, gsem, osem):
    wid = lax.axis_index("s") * NC + lax.axis_index("c")
    base = wid * BPW
    pltpu.sync_copy(ori_hbm.at[pl.ds(base, BPW)], ori_v)

    def stage_indices(q, b):
        for g in range(CB // LANES):
            o = ori_v[pl.ds(q * CB + g * LANES, LANES)]
            s = (o + 1.0) * 0.5 * float(N_EMBED)
            i0 = s.astype(jnp.int32)
            f = s - i0.astype(jnp.float32)
            il = jnp.where(i0 >= N_EMBED, i0 - N_EMBED, i0)
            i1 = i0 + 1
            ir = jnp.where(i1 >= N_EMBED, i1 - N_EMBED, i1)
            sl = pl.ds(g * LANES, LANES)
            idxc[b][sl] = il
            idxc[b][pl.ds(CB + g * LANES, LANES)] = ir
            wl_v[b][sl] = 1.0 - f
            wr_v[b][sl] = f

    def start_gather(b):
        pltpu.async_copy(table_hbm.at[idxc[b]], bufc[b], gsem[b])

    def wait_gather(b):
        pltpu.make_async_copy(table_hbm.at[idxc[b]], bufc[b], gsem[b]).wait()

    def interp_chunk(b):
        def row_body(j, carry):
            jv = jnp.zeros((LANES,), jnp.int32) + j
            wl = plsc.load_gather(wl_v[b], [jv])
            wr = plsc.load_gather(wr_v[b], [jv])
            for v in range(DS // LANES):
                sl = pl.ds(v * LANES, LANES)
                bufc[b][j, sl] = bufc[b][j, sl] * wl + bufc[b][CB + j, sl] * wr
            return carry

        lax.fori_loop(0, CB, row_body, 0, unroll=2)

    stage_indices(0, 0)
    start_gather(0)

    def wait_outcopy(b):
        pltpu.make_async_copy(
            bufc[b].at[pl.ds(0, CB)],
            out_hbm.at[pl.ds(base, CB)], osem[b]).wait()

    def outer(q0, carry):
        for b in range(NBUF):
            q = q0 + b

            @pl.when(q + 1 < NCHUNK)
            def _():
                @pl.when(q >= 1)
                def _():
                    wait_outcopy(1 - b)

                stage_indices(q + 1, 1 - b)
                start_gather(1 - b)

            wait_gather(b)
            interp_chunk(b)
            pltpu.async_copy(
                bufc[b].at[pl.ds(0, CB)],
                out_hbm.at[pl.ds(base + q * CB, CB)], osem[b])
        return carry

    lax.fori_loop(0, NCHUNK // NBUF, lambda i, c: outer(i * NBUF, c), 0)

    for b in range(NBUF):
        wait_outcopy(b)


def _transpose_body(x_ref, o_ref):
    x = x_ref[...]
    o_ref[...] = jnp.transpose(x.reshape(DS, BN), (1, 0))


def _make_transpose(si):
    grid = (N_EMBED + BN - 1) // BN
    return pl.pallas_call(
        _transpose_body,
        grid=(grid,),
        in_specs=[pl.BlockSpec((LPS, CH, BN), lambda i, si=si: (si, 0, i))],
        out_specs=pl.BlockSpec((BN, DS), lambda i: (i, 0)),
        out_shape=jax.ShapeDtypeStruct((N_EMBED, DS), jnp.float32),
    )


def kernel(ori, embeds):
    et = embeds.transpose(1, 2, 0)
    outs = []
    for si in range(S):
        table_s = _make_transpose(si)(et)
        outs.append(_embed_slice(ori, table_s))
    out = jnp.concatenate(outs, axis=1)
    return out.reshape(B_TOT, N_LAYER, CH)


# speedup vs baseline: 2.4518x; 1.1743x over previous
import functools
import jax
import jax.numpy as jnp
from jax import lax
from jax.experimental import pallas as pl
from jax.experimental.pallas import tpu as pltpu
from jax.experimental.pallas import tpu_sc as plsc

N_EMBED = 100000
N_LAYER = 16
CH = 64
D = N_LAYER * CH
B_TOT = 16384
S = 4
LPS = N_LAYER // S
DS = LPS * CH
NC, NS, LANES = 2, 16, 16
NW = NC * NS
BPW = B_TOT // NW
CB = 64
NCHUNK = BPW // CB
NBUF = 2
BN = 8192

_mesh = plsc.VectorSubcoreMesh(core_axis_name="c", subcore_axis_name="s")


@functools.partial(
    pl.kernel,
    out_type=jax.ShapeDtypeStruct((B_TOT, DS), jnp.float32),
    mesh=_mesh,
    scratch_types=[
        pltpu.VMEM((BPW,), jnp.float32),
        [pltpu.VMEM((2 * CB,), jnp.int32) for _ in range(NBUF)],
        [pltpu.VMEM((CB,), jnp.float32) for _ in range(NBUF)],
        [pltpu.VMEM((CB,), jnp.float32) for _ in range(NBUF)],
        [pltpu.VMEM((2 * CB, DS), jnp.float32) for _ in range(NBUF)],
        [pltpu.SemaphoreType.DMA for _ in range(NBUF)],
        [pltpu.SemaphoreType.DMA for _ in range(NBUF)],
    ],
    compiler_params=pltpu.CompilerParams(needs_layout_passes=False),
)
def _embed_slice(ori_hbm, table_hbm, out_hbm,
                 ori_v, idxc, wl_v, wr_v, bufc, gsem, osem):
    wid = lax.axis_index("s") * NC + lax.axis_index("c")
    base = wid * BPW
    pltpu.sync_copy(ori_hbm.at[pl.ds(base, BPW)], ori_v)

    def stage_indices(q, b):
        for g in range(CB // LANES):
            o = ori_v[pl.ds(q * CB + g * LANES, LANES)]
            s = (o + 1.0) * 0.5 * float(N_EMBED)
            i0 = s.astype(jnp.int32)
            f = s - i0.astype(jnp.float32)
            il = jnp.where(i0 >= N_EMBED, i0 - N_EMBED, i0)
            i1 = i0 + 1
            ir = jnp.where(i1 >= N_EMBED, i1 - N_EMBED, i1)
            sl = pl.ds(g * LANES, LANES)
            idxc[b][sl] = il
            idxc[b][pl.ds(CB + g * LANES, LANES)] = ir
            wl_v[b][sl] = 1.0 - f
            wr_v[b][sl] = f

    def start_gather(b):
        pltpu.async_copy(table_hbm.at[idxc[b]], bufc[b], gsem[b])

    def wait_gather(b):
        pltpu.make_async_copy(table_hbm.at[idxc[b]], bufc[b], gsem[b]).wait()

    def interp_chunk(b):
        def row_body(j, carry):
            jv = jnp.zeros((LANES,), jnp.int32) + j
            wl = plsc.load_gather(wl_v[b], [jv])
            wr = plsc.load_gather(wr_v[b], [jv])
            for v in range(DS // LANES):
                sl = pl.ds(v * LANES, LANES)
                bufc[b][j, sl] = bufc[b][j, sl] * wl + bufc[b][CB + j, sl] * wr
            return carry

        lax.fori_loop(0, CB, row_body, 0, unroll=2)

    stage_indices(0, 0)
    start_gather(0)

    def wait_outcopy(b):
        pltpu.make_async_copy(
            bufc[b].at[pl.ds(0, CB)],
            out_hbm.at[pl.ds(base, CB)], osem[b]).wait()

    def outer(q0, carry):
        for b in range(NBUF):
            q = q0 + b

            @pl.when(q + 1 < NCHUNK)
            def _():
                @pl.when(q >= 1)
                def _():
                    wait_outcopy(1 - b)

                stage_indices(q + 1, 1 - b)
                start_gather(1 - b)

            wait_gather(b)
            interp_chunk(b)
            pltpu.async_copy(
                bufc[b].at[pl.ds(0, CB)],
                out_hbm.at[pl.ds(base + q * CB, CB)], osem[b])
        return carry

    lax.fori_loop(0, NCHUNK // NBUF, lambda i, c: outer(i * NBUF, c), 0)

    for b in range(NBUF):
        wait_outcopy(b)


def _transpose_body(x_ref, o_ref):
    x = x_ref[...]
    o_ref[...] = jnp.transpose(x.reshape(DS, BN), (1, 0))


def _make_transpose(si):
    grid = (N_EMBED + BN - 1) // BN
    return pl.pallas_call(
        _transpose_body,
        grid=(grid,),
        in_specs=[pl.BlockSpec((LPS, CH, BN), lambda i, si=si: (si, 0, i))],
        out_specs=pl.BlockSpec((BN, DS), lambda i: (i, 0)),
        out_shape=jax.ShapeDtypeStruct((N_EMBED, DS), jnp.float32),
    )


BT = 4096


def _out_transpose_body(x_ref, o_ref):
    o_ref[...] = jnp.transpose(x_ref[...], (1, 0))


def _out_transpose_alias_body(x_ref, acc_ref, o_ref):
    del acc_ref
    o_ref[...] = jnp.transpose(x_ref[...], (1, 0))


def _make_out_transpose(si):
    out_shape = jax.ShapeDtypeStruct((D, B_TOT), jnp.float32)
    out_spec = pl.BlockSpec((DS, BT), lambda i, si=si: (si, i))
    in_spec = pl.BlockSpec((BT, DS), lambda i: (i, 0))
    if si == 0:
        return pl.pallas_call(
            _out_transpose_body,
            grid=(B_TOT // BT,),
            in_specs=[in_spec],
            out_specs=out_spec,
            out_shape=out_shape,
        )
    return pl.pallas_call(
        _out_transpose_alias_body,
        grid=(B_TOT // BT,),
        in_specs=[in_spec, pl.BlockSpec(memory_space=pl.ANY)],
        out_specs=out_spec,
        out_shape=out_shape,
        input_output_aliases={1: 0},
    )


def kernel(ori, embeds):
    et = embeds.transpose(1, 2, 0)
    acc = None
    for si in range(S):
        table_s = _make_transpose(si)(et)
        piece = _embed_slice(ori, table_s)
        if si == 0:
            acc = _make_out_transpose(0)(piece)
        else:
            acc = _make_out_transpose(si)(piece, acc)
    return acc.reshape(N_LAYER, CH, B_TOT).transpose(2, 0, 1)


# speedup vs baseline: 2.4887x; 1.0151x over previous
import functools
import jax
import jax.numpy as jnp
from jax import lax
from jax.experimental import pallas as pl
from jax.experimental.pallas import tpu as pltpu
from jax.experimental.pallas import tpu_sc as plsc

N_EMBED = 100000
N_LAYER = 16
CH = 64
D = N_LAYER * CH
B_TOT = 16384
S = 4
LPS = N_LAYER // S
DS = LPS * CH
NC, NS, LANES = 2, 16, 16
NW = NC * NS
BPW = B_TOT // NW
CB = 64
NCHUNK = BPW // CB
NBUF = 2
BN = 8192

_mesh = plsc.VectorSubcoreMesh(core_axis_name="c", subcore_axis_name="s")


@functools.partial(
    pl.kernel,
    out_type=jax.ShapeDtypeStruct((B_TOT, DS), jnp.float32),
    mesh=_mesh,
    scratch_types=[
        pltpu.VMEM((BPW,), jnp.float32),
        [pltpu.VMEM((2 * CB,), jnp.int32) for _ in range(NBUF)],
        [pltpu.VMEM((CB,), jnp.float32) for _ in range(NBUF)],
        [pltpu.VMEM((CB,), jnp.float32) for _ in range(NBUF)],
        [pltpu.VMEM((2 * CB, DS), jnp.float32) for _ in range(NBUF)],
        [pltpu.SemaphoreType.DMA for _ in range(NBUF)],
        [pltpu.SemaphoreType.DMA for _ in range(NBUF)],
    ],
    compiler_params=pltpu.CompilerParams(needs_layout_passes=False),
)
def _embed_slice(ori_hbm, table_hbm, out_hbm,
                 ori_v, idxc, wl_v, wr_v, bufc, gsem, osem):
    wid = lax.axis_index("s") * NC + lax.axis_index("c")
    base = wid * BPW
    pltpu.sync_copy(ori_hbm.at[pl.ds(base, BPW)], ori_v)

    def stage_indices(q, b):
        for g in range(CB // LANES):
            o = ori_v[pl.ds(q * CB + g * LANES, LANES)]
            s = (o + 1.0) * 0.5 * float(N_EMBED)
            i0 = s.astype(jnp.int32)
            f = s - i0.astype(jnp.float32)
            il = jnp.where(i0 >= N_EMBED, i0 - N_EMBED, i0)
            i1 = i0 + 1
            ir = jnp.where(i1 >= N_EMBED, i1 - N_EMBED, i1)
            sl = pl.ds(g * LANES, LANES)
            idxc[b][sl] = il
            idxc[b][pl.ds(CB + g * LANES, LANES)] = ir
            wl_v[b][sl] = 1.0 - f
            wr_v[b][sl] = f

    def start_gather(b):
        pltpu.async_copy(table_hbm.at[idxc[b]], bufc[b], gsem[b])

    def wait_gather(b):
        pltpu.make_async_copy(table_hbm.at[idxc[b]], bufc[b], gsem[b]).wait()

    def interp_chunk(b):
        def row_body(j, carry):
            jv = jnp.zeros((LANES,), jnp.int32) + j
            wl = plsc.load_gather(wl_v[b], [jv])
            wr = plsc.load_gather(wr_v[b], [jv])
            for v in range(DS // LANES):
                sl = pl.ds(v * LANES, LANES)
                bufc[b][j, sl] = bufc[b][j, sl] * wl + bufc[b][CB + j, sl] * wr
            return carry

        lax.fori_loop(0, CB, row_body, 0, unroll=2)

    stage_indices(0, 0)
    start_gather(0)

    def wait_outcopy(b):
        pltpu.make_async_copy(
            bufc[b].at[pl.ds(0, CB)],
            out_hbm.at[pl.ds(base, CB)], osem[b]).wait()

    def outer(q0, carry):
        for b in range(NBUF):
            q = q0 + b

            @pl.when(q + 1 < NCHUNK)
            def _():
                @pl.when(q >= 1)
                def _():
                    wait_outcopy(1 - b)

                stage_indices(q + 1, 1 - b)
                start_gather(1 - b)

            wait_gather(b)
            interp_chunk(b)
            pltpu.async_copy(
                bufc[b].at[pl.ds(0, CB)],
                out_hbm.at[pl.ds(base + q * CB, CB)], osem[b])
        return carry

    lax.fori_loop(0, NCHUNK // NBUF, lambda i, c: outer(i * NBUF, c), 0)

    for b in range(NBUF):
        wait_outcopy(b)


def _transpose_body(x_ref, o_ref):
    x = x_ref[...]
    o_ref[...] = jnp.transpose(x.reshape(DS, BN), (1, 0))


def _make_transpose(si):
    grid = (N_EMBED + BN - 1) // BN
    return pl.pallas_call(
        _transpose_body,
        grid=(grid,),
        in_specs=[pl.BlockSpec((LPS, CH, BN), lambda i, si=si: (si, 0, i))],
        out_specs=pl.BlockSpec((BN, DS), lambda i: (i, 0)),
        out_shape=jax.ShapeDtypeStruct((N_EMBED, DS), jnp.float32),
    )


BT = 8192


def _out_transpose_body(x_ref, o_ref):
    o_ref[...] = jnp.transpose(x_ref[...], (1, 0))


def _out_transpose_alias_body(x_ref, acc_ref, o_ref):
    del acc_ref
    o_ref[...] = jnp.transpose(x_ref[...], (1, 0))


def _make_out_transpose(si):
    out_shape = jax.ShapeDtypeStruct((D, B_TOT), jnp.float32)
    out_spec = pl.BlockSpec((DS, BT), lambda i, si=si: (si, i))
    in_spec = pl.BlockSpec((BT, DS), lambda i: (i, 0))
    if si == 0:
        return pl.pallas_call(
            _out_transpose_body,
            grid=(B_TOT // BT,),
            in_specs=[in_spec],
            out_specs=out_spec,
            out_shape=out_shape,
        )
    return pl.pallas_call(
        _out_transpose_alias_body,
        grid=(B_TOT // BT,),
        in_specs=[in_spec, pl.BlockSpec(memory_space=pl.ANY)],
        out_specs=out_spec,
        out_shape=out_shape,
        input_output_aliases={1: 0},
    )


def kernel(ori, embeds):
    et = embeds.transpose(1, 2, 0)
    acc = None
    for si in range(S):
        table_s = _make_transpose(si)(et)
        piece = _embed_slice(ori, table_s)
        if si == 0:
            acc = _make_out_transpose(0)(piece)
        else:
            acc = _make_out_transpose(si)(piece, acc)
    return acc.reshape(N_LAYER, CH, B_TOT).transpose(2, 0, 1)
